# Initial kernel scaffold; baseline (speedup 1.0000x reference)
import jax
import jax.numpy as jnp
from jax.experimental import pallas as pl


def kernel(hidden_states, Wg, W1, W3, W2):
    raise NotImplementedError("write your pallas kernel here")


# speedup vs baseline: 1.1499x; 1.1499x over previous
import functools

import jax
import jax.numpy as jnp
from jax import lax
from jax.experimental import pallas as pl
from jax.experimental.pallas import tpu as pltpu

_JITTER = 0.01
_NEG_INF = float("-inf")


def _router_body(x_ref, wg_ref, logits_ref, w_ref):
    x = x_ref[...]
    wg = wg_ref[...]
    s = lax.dot_general(x, wg, (((1,), (1,)), ((), ())),
                        preferred_element_type=jnp.float32)
    logits_ref[...] = s
    bt, e = s.shape
    iota = lax.broadcasted_iota(jnp.int32, (bt, e), 1)
    t1 = jnp.max(s, axis=-1, keepdims=True)
    i1 = jnp.min(jnp.where(s == t1, iota, e), axis=-1, keepdims=True)
    oh1 = iota == i1
    factor1 = jnp.maximum(jnp.abs(s), t1)
    mask1 = ((t1 - s) / factor1) > (2.0 * _JITTER)
    z1 = jnp.where(mask1, _NEG_INF, s)
    a1 = jnp.exp(z1 - t1)
    m1 = jnp.sum(jnp.where(oh1, a1, 0.0), axis=-1, keepdims=True) / jnp.sum(
        a1, axis=-1, keepdims=True)
    s2 = jnp.where(oh1, _NEG_INF, s)
    t2 = jnp.max(s2, axis=-1, keepdims=True)
    i2 = jnp.min(jnp.where(s2 == t2, iota, e), axis=-1, keepdims=True)
    oh2 = iota == i2
    factor2 = jnp.maximum(jnp.abs(s), t2)
    mask2 = ((t2 - s) / factor2) > (2.0 * _JITTER)
    z2 = jnp.where(mask2, _NEG_INF, s2)
    a2 = jnp.exp(z2 - t2)
    m2 = jnp.sum(jnp.where(oh2, a2, 0.0), axis=-1, keepdims=True) / jnp.sum(
        a2, axis=-1, keepdims=True)
    w_ref[...] = jnp.where(oh1, m1, 0.0) + jnp.where(oh2, m2, 0.0)


def _moe_body(x_ref, w_ref, w1_ref, w3_ref, w2_ref, out_ref):
    e = pl.program_id(1)
    x = x_ref[...]
    h1 = lax.dot_general(x, w1_ref[0], (((1,), (1,)), ((), ())),
                         preferred_element_type=jnp.float32)
    h3 = lax.dot_general(x, w3_ref[0], (((1,), (1,)), ((), ())),
                         preferred_element_type=jnp.float32)
    h = (h1 * jax.nn.sigmoid(h1)) * h3
    y = lax.dot_general(h, w2_ref[0], (((1,), (1,)), ((), ())),
                        preferred_element_type=jnp.float32)
    w = w_ref[...]
    lane = lax.broadcasted_iota(jnp.int32, w.shape, 1)
    wcol = jnp.sum(jnp.where(lane == e, w, 0.0), axis=-1, keepdims=True)
    y = y * wcol

    @pl.when(e == 0)
    def _():
        out_ref[...] = y

    @pl.when(e > 0)
    def _():
        out_ref[...] += y


@jax.jit
def kernel(hidden_states, Wg, W1, W3, W2):
    b, s, d = hidden_states.shape
    t = b * s
    e, dff, _ = W1.shape
    x = hidden_states.reshape(t, d)
    bt = 256

    logits, w = pl.pallas_call(
        _router_body,
        grid=(t // bt,),
        in_specs=[
            pl.BlockSpec((bt, d), lambda i: (i, 0)),
            pl.BlockSpec((e, d), lambda i: (0, 0)),
        ],
        out_specs=[
            pl.BlockSpec((bt, e), lambda i: (i, 0)),
            pl.BlockSpec((bt, e), lambda i: (i, 0)),
        ],
        out_shape=[
            jax.ShapeDtypeStruct((t, e), jnp.float32),
            jax.ShapeDtypeStruct((t, e), jnp.float32),
        ],
    )(x, Wg)

    out = pl.pallas_call(
        _moe_body,
        grid=(t // bt, e),
        in_specs=[
            pl.BlockSpec((bt, d), lambda i, j: (i, 0)),
            pl.BlockSpec((bt, e), lambda i, j: (i, 0)),
            pl.BlockSpec((1, dff, d), lambda i, j: (j, 0, 0)),
            pl.BlockSpec((1, dff, d), lambda i, j: (j, 0, 0)),
            pl.BlockSpec((1, d, dff), lambda i, j: (j, 0, 0)),
        ],
        out_specs=pl.BlockSpec((bt, d), lambda i, j: (i, 0)),
        out_shape=jax.ShapeDtypeStruct((t, d), jnp.float32),
        compiler_params=pltpu.CompilerParams(
            dimension_semantics=("parallel", "arbitrary")),
    )(x, w, W1, W3, W2)

    return out.reshape(b, s, d), logits
